# baseline (speedup 1.0000x reference)
import functools

import jax
import jax.numpy as jnp
from jax import lax
from jax.experimental import pallas as pl
from jax.experimental.pallas import tpu as pltpu
from jax.experimental.pallas import tpu_sc as plsc

_N_USERS = 5000
_N_ITEMS = 5000
_N = _N_USERS + _N_ITEMS
_NPAD = 10240
_B = 4096
_F = 128
_H = 64
_NC = 2
_NS = 16
_NW = _NC * _NS
_CHUNK = _NPAD // _NW
_BW = _B // _NW

_sc_mesh = plsc.VectorSubcoreMesh(
    core_axis_name="c", subcore_axis_name="s",
    num_cores=_NC, num_subcores=_NS)


def _mask_body(rows_hbm, mask_hbm, idx_v, local_v):
    wid = lax.axis_index("s") * _NC + lax.axis_index("c")
    base = wid * _CHUNK
    zero16 = jnp.zeros((16,), jnp.float32)
    for k in range(_CHUNK // 16):
        local_v[pl.ds(k * 16, 16)] = zero16
    pltpu.sync_copy(rows_hbm, idx_v)
    ones16 = jnp.ones((16,), jnp.float32)

    def body(k, carry):
        idx = idx_v[pl.ds(k * 16, 16)]
        rel = idx - base
        inb = (rel >= 0) & (rel < _CHUNK)
        relc = jnp.clip(rel, 0, _CHUNK - 1)
        plsc.store_scatter(local_v, [relc], ones16, mask=inb)
        return carry

    lax.fori_loop(0, (2 * _B) // 16, body, 0)
    pltpu.sync_copy(local_v, mask_hbm.at[pl.ds(base, _CHUNK)])


_mask_call = functools.partial(
    pl.kernel,
    out_type=jax.ShapeDtypeStruct((_NPAD,), jnp.float32),
    mesh=_sc_mesh,
    compiler_params=pltpu.CompilerParams(needs_layout_passes=False),
    scratch_types=[
        pltpu.VMEM((2 * _B,), jnp.int32),
        pltpu.VMEM((_CHUNK,), jnp.float32),
    ],
)(_mask_body)


def _support1_body(u_ref, i_ref, w1_ref, b1_ref, mask_ref, out_ref):
    w1 = w1_ref[...]
    dn = (((1,), (1,)), ((), ()))
    tu = lax.dot_general(u_ref[...].astype(jnp.bfloat16), w1, dn,
                         preferred_element_type=jnp.float32)
    ti = lax.dot_general(i_ref[...].astype(jnp.bfloat16), w1, dn,
                         preferred_element_type=jnp.float32)
    t = jnp.concatenate([tu, ti], axis=0)
    s1 = t * mask_ref[...] + b1_ref[...]
    out_ref[...] = s1.astype(jnp.bfloat16)


def _support1(user_table, item_table, w1b, b1r, mask2d):
    return pl.pallas_call(
        _support1_body,
        out_shape=jax.ShapeDtypeStruct((_N, _H), jnp.bfloat16),
    )(user_table, item_table, w1b, b1r, mask2d)


_RB = 400


def _pass1_body(a_ref, s1_ref, w2_ref, b2_ref, out_ref):
    a = a_ref[...].astype(jnp.bfloat16)
    h = lax.dot_general(a, s1_ref[...], (((1,), (0,)), ((), ())),
                        preferred_element_type=jnp.float32)
    h = jnp.maximum(h, 0.0).astype(jnp.bfloat16)
    s2 = lax.dot_general(h, w2_ref[...], (((1,), (1,)), ((), ())),
                         preferred_element_type=jnp.float32) + b2_ref[...]
    out_ref[...] = s2.astype(jnp.bfloat16)


def _pass1(adj, s1, w2b, b2r):
    return pl.pallas_call(
        _pass1_body,
        grid=(_N // _RB,),
        in_specs=[
            pl.BlockSpec((_RB, _N), lambda i: (i, 0)),
            pl.BlockSpec((_N, _H), lambda i: (0, 0)),
            pl.BlockSpec((_H, _H), lambda i: (0, 0)),
            pl.BlockSpec((1, _H), lambda i: (0, 0)),
        ],
        out_specs=pl.BlockSpec((_RB, _H), lambda i: (i, 0)),
        out_shape=jax.ShapeDtypeStruct((_N, _H), jnp.bfloat16),
    )(adj, s1, w2b, b2r)


_M2 = 64
_NROWS2 = 2 * _B


def _pass2_body(ids_ref, k_ref, a_any, s2_ref, wfc2_ref, out_ref,
                buf0, buf1, sem0, sem1):
    i = pl.program_id(0)
    k = k_ref[0]
    even = lax.rem(i, 2) == 0

    def issue(prog, buf, sem):
        for j in range(_M2):
            rid = ids_ref[prog * _M2 + j]
            pltpu.make_async_copy(
                a_any.at[pl.ds(rid, 1)], buf.at[pl.ds(j, 1)], sem).start()

    def drain_compute(prog, buf, sem):
        for j in range(_M2):
            rid = ids_ref[prog * _M2 + j]
            pltpu.make_async_copy(
                a_any.at[pl.ds(rid, 1)], buf.at[pl.ds(j, 1)], sem).wait()
        a = buf[...].astype(jnp.bfloat16)
        o = lax.dot_general(a, s2_ref[...], (((1,), (0,)), ((), ())),
                            preferred_element_type=jnp.float32)
        o = jnp.maximum(o, 0.0)
        out_ref[...] = jnp.concatenate([o, o], axis=1) * wfc2_ref[...]

    @pl.when(i == 0)
    def _():
        issue(0, buf0, sem0)

    nxt_valid = (i + 1) * _M2 < k

    @pl.when(nxt_valid & even)
    def _():
        issue(i + 1, buf1, sem1)

    @pl.when(nxt_valid & jnp.logical_not(even))
    def _():
        issue(i + 1, buf0, sem0)

    cur_valid = i * _M2 < k

    @pl.when(cur_valid & even)
    def _():
        drain_compute(i, buf0, sem0)

    @pl.when(cur_valid & jnp.logical_not(even))
    def _():
        drain_compute(i, buf1, sem1)


def _pass2(adj, s2, wfc2, ids, kk):
    return pl.pallas_call(
        _pass2_body,
        grid_spec=pltpu.PrefetchScalarGridSpec(
            num_scalar_prefetch=2,
            grid=(_NROWS2 // _M2,),
            in_specs=[
                pl.BlockSpec(memory_space=pltpu.MemorySpace.HBM),
                pl.BlockSpec((_N, _H), lambda i, *_: (0, 0)),
                pl.BlockSpec((1, 2 * _H), lambda i, *_: (0, 0)),
            ],
            out_specs=pl.BlockSpec((_M2, 2 * _H), lambda i, *_: (i, 0)),
            scratch_shapes=[
                pltpu.VMEM((_M2, _N), jnp.float32),
                pltpu.VMEM((_M2, _N), jnp.float32),
                pltpu.SemaphoreType.DMA,
                pltpu.SemaphoreType.DMA,
            ],
        ),
        out_shape=jax.ShapeDtypeStruct((_NROWS2, 2 * _H), jnp.float32),
    )(ids, kk, adj, s2, wfc2)


def _rate_body(feat_hbm, uidx_hbm, iidx_hbm, bfc_hbm, rating_hbm,
               uidx_v, iidx_v, uro_v, iro_v, rat_v, bfc_v, sem1, sem2):
    wid = lax.axis_index("s") * _NC + lax.axis_index("c")
    base = wid * _BW
    pltpu.sync_copy(uidx_hbm.at[pl.ds(base, _BW)], uidx_v)
    pltpu.sync_copy(iidx_hbm.at[pl.ds(base, _BW)], iidx_v)
    pltpu.sync_copy(bfc_hbm, bfc_v)
    cp1 = pltpu.async_copy(feat_hbm.at[uidx_v], uro_v, sem1)
    cp2 = pltpu.async_copy(feat_hbm.at[iidx_v], iro_v, sem2)
    cp1.wait()
    cp2.wait()
    bfc16 = bfc_v[...]

    def g_body(g, carry):
        b0 = g * 16
        bidx = b0 + lax.iota(jnp.int32, 16)
        acc = bfc16
        for f in range(_H):
            fidx = jnp.full((16,), f, jnp.int32)
            fidx2 = jnp.full((16,), _H + f, jnp.int32)
            u = plsc.load_gather(uro_v, [bidx, fidx])
            iv = plsc.load_gather(iro_v, [bidx, fidx2])
            acc = acc + u * iv
        rat_v[pl.ds(b0, 16)] = acc
        return carry

    lax.fori_loop(0, _BW // 16, g_body, 0)
    pltpu.sync_copy(rat_v, rating_hbm.at[pl.ds(base, _BW)])


_rate_call = functools.partial(
    pl.kernel,
    out_type=jax.ShapeDtypeStruct((_B,), jnp.float32),
    mesh=_sc_mesh,
    compiler_params=pltpu.CompilerParams(needs_layout_passes=False),
    scratch_types=[
        pltpu.VMEM((_BW,), jnp.int32),
        pltpu.VMEM((_BW,), jnp.int32),
        pltpu.VMEM((_BW, 2 * _H), jnp.float32),
        pltpu.VMEM((_BW, 2 * _H), jnp.float32),
        pltpu.VMEM((_BW,), jnp.float32),
        pltpu.VMEM((16,), jnp.float32),
        pltpu.SemaphoreType.DMA,
        pltpu.SemaphoreType.DMA,
    ],
)(_rate_body)


def kernel(user_indices, item_indices, adjacency_matrix, user_table,
           item_table, W1, b1, W2, b2, Wfc, bfc):
    ui = user_indices.astype(jnp.int32)
    ii = item_indices.astype(jnp.int32) + _N_USERS
    rows_all = jnp.concatenate([ui, ii])
    mask = _mask_call(rows_all)
    mask2d = mask[:_N].reshape(_N, 1)
    mi = mask[:_N].astype(jnp.int32)
    cnt = jnp.cumsum(mi)
    kk = cnt[_N - 1:]
    pos = cnt - 1
    last_id = jnp.argmax(cnt).astype(jnp.int32)
    ids = jnp.full((_NROWS2,), last_id, jnp.int32)
    ids = ids.at[jnp.where(mi == 1, pos, _NROWS2 * 4)].set(
        jnp.arange(_N, dtype=jnp.int32), mode="drop")
    s1 = _support1(user_table, item_table, W1.astype(jnp.bfloat16),
                   b1.reshape(1, _H), mask2d)
    s2 = _pass1(adjacency_matrix, s1, W2.astype(jnp.bfloat16),
                b2.reshape(1, _H))
    wfc2 = jnp.concatenate(
        [Wfc.reshape(1, _H), jnp.ones((1, _H), jnp.float32)], axis=1)
    feat = _pass2(adjacency_matrix, s2, wfc2, ids, kk)
    rating = _rate_call(feat, pos[ui], pos[ii],
                        jnp.broadcast_to(bfc, (16,)).astype(jnp.float32))
    return rating


# speedup vs baseline: 1.1596x; 1.1596x over previous
import functools

import jax
import jax.numpy as jnp
from jax import lax
from jax.experimental import pallas as pl
from jax.experimental.pallas import tpu as pltpu
from jax.experimental.pallas import tpu_sc as plsc

_N_USERS = 5000
_N_ITEMS = 5000
_N = _N_USERS + _N_ITEMS
_NPAD = 10240
_B = 4096
_F = 128
_H = 64
_NC = 2
_NS = 16
_NW = _NC * _NS
_CHUNK = _NPAD // _NW
_BW = _B // _NW

_sc_mesh = plsc.VectorSubcoreMesh(
    core_axis_name="c", subcore_axis_name="s",
    num_cores=_NC, num_subcores=_NS)


def _mask_body(rows_hbm, mask_hbm, idx_v, local_v):
    wid = lax.axis_index("s") * _NC + lax.axis_index("c")
    base = wid * _CHUNK
    zero16 = jnp.zeros((16,), jnp.float32)
    for k in range(_CHUNK // 16):
        local_v[pl.ds(k * 16, 16)] = zero16
    pltpu.sync_copy(rows_hbm, idx_v)
    ones16 = jnp.ones((16,), jnp.float32)

    def body(k, carry):
        idx = idx_v[pl.ds(k * 16, 16)]
        rel = idx - base
        inb = (rel >= 0) & (rel < _CHUNK)
        relc = jnp.clip(rel, 0, _CHUNK - 1)
        plsc.store_scatter(local_v, [relc], ones16, mask=inb)
        return carry

    lax.fori_loop(0, (2 * _B) // 16, body, 0)
    pltpu.sync_copy(local_v, mask_hbm.at[pl.ds(base, _CHUNK)])


_mask_call = functools.partial(
    pl.kernel,
    out_type=jax.ShapeDtypeStruct((_NPAD,), jnp.float32),
    mesh=_sc_mesh,
    compiler_params=pltpu.CompilerParams(needs_layout_passes=False),
    scratch_types=[
        pltpu.VMEM((2 * _B,), jnp.int32),
        pltpu.VMEM((_CHUNK,), jnp.float32),
    ],
)(_mask_body)


def _support1_body(u_ref, i_ref, w1_ref, b1_ref, mask_ref, out_ref):
    w1 = w1_ref[...]
    dn = (((1,), (1,)), ((), ()))
    tu = lax.dot_general(u_ref[...].astype(jnp.bfloat16), w1, dn,
                         preferred_element_type=jnp.float32)
    ti = lax.dot_general(i_ref[...].astype(jnp.bfloat16), w1, dn,
                         preferred_element_type=jnp.float32)
    t = jnp.concatenate([tu, ti], axis=0)
    s1 = t * mask_ref[...] + b1_ref[...]
    out_ref[...] = s1.astype(jnp.bfloat16)


def _support1(user_table, item_table, w1b, b1r, mask2d):
    return pl.pallas_call(
        _support1_body,
        out_shape=jax.ShapeDtypeStruct((_N, _H), jnp.bfloat16),
    )(user_table, item_table, w1b, b1r, mask2d)


_RB = 400


def _pass1_body(a_ref, s1_ref, w2_ref, b2_ref, out_ref):
    a = a_ref[...].astype(jnp.bfloat16)
    h = lax.dot_general(a, s1_ref[...], (((1,), (0,)), ((), ())),
                        preferred_element_type=jnp.float32)
    h = jnp.maximum(h, 0.0).astype(jnp.bfloat16)
    s2 = lax.dot_general(h, w2_ref[...], (((1,), (1,)), ((), ())),
                         preferred_element_type=jnp.float32) + b2_ref[...]
    out_ref[...] = s2.astype(jnp.bfloat16)


def _pass1(adj, s1, w2b, b2r):
    return pl.pallas_call(
        _pass1_body,
        grid=(_N // _RB,),
        in_specs=[
            pl.BlockSpec((_RB, _N), lambda i: (i, 0)),
            pl.BlockSpec((_N, _H), lambda i: (0, 0)),
            pl.BlockSpec((_H, _H), lambda i: (0, 0)),
            pl.BlockSpec((1, _H), lambda i: (0, 0)),
        ],
        out_specs=pl.BlockSpec((_RB, _H), lambda i: (i, 0)),
        out_shape=jax.ShapeDtypeStruct((_N, _H), jnp.bfloat16),
    )(adj, s1, w2b, b2r)


_M2 = 128
_NROWS2 = 2 * _B
_NSLOT = 4


def _pass2_body(ids_ref, k_ref, a_any, s2_ref, wfc2_ref, out_ref, *scratch):
    bufs = scratch[:_NSLOT]
    sems = scratch[_NSLOT:]
    i = pl.program_id(0)
    k = k_ref[0]
    r = lax.rem(i, _NSLOT)

    def issue(prog, buf, sem):
        for j in range(_M2):
            rid = ids_ref[prog * _M2 + j]
            pltpu.make_async_copy(
                a_any.at[pl.ds(rid, 1)], buf.at[pl.ds(j, 1)], sem).start()

    def drain_compute(prog, buf, sem):
        for j in range(_M2):
            rid = ids_ref[prog * _M2 + j]
            pltpu.make_async_copy(
                a_any.at[pl.ds(rid, 1)], buf.at[pl.ds(j, 1)], sem).wait()
        a = buf[...].astype(jnp.bfloat16)
        o = lax.dot_general(a, s2_ref[...], (((1,), (0,)), ((), ())),
                            preferred_element_type=jnp.float32)
        o = jnp.maximum(o, 0.0)
        out_ref[...] = jnp.concatenate([o, o], axis=1) * wfc2_ref[...]

    for p in range(_NSLOT - 1):
        @pl.when((i == 0) & (p * _M2 < k))
        def _(p=p):
            issue(p, bufs[p], sems[p])

    nxt = i + _NSLOT - 1
    nxt_valid = nxt * _M2 < k
    for s in range(_NSLOT):
        tgt = (s + _NSLOT - 1) % _NSLOT

        @pl.when(nxt_valid & (r == s))
        def _(tgt=tgt):
            issue(nxt, bufs[tgt], sems[tgt])

    cur_valid = i * _M2 < k
    for s in range(_NSLOT):
        @pl.when(cur_valid & (r == s))
        def _(s=s):
            drain_compute(i, bufs[s], sems[s])


def _pass2(adj, s2, wfc2, ids, kk):
    return pl.pallas_call(
        _pass2_body,
        grid_spec=pltpu.PrefetchScalarGridSpec(
            num_scalar_prefetch=2,
            grid=(_NROWS2 // _M2,),
            in_specs=[
                pl.BlockSpec(memory_space=pltpu.MemorySpace.HBM),
                pl.BlockSpec((_N, _H), lambda i, *_: (0, 0)),
                pl.BlockSpec((1, 2 * _H), lambda i, *_: (0, 0)),
            ],
            out_specs=pl.BlockSpec((_M2, 2 * _H), lambda i, *_: (i, 0)),
            scratch_shapes=(
                [pltpu.VMEM((_M2, _N), jnp.float32)] * _NSLOT
                + [pltpu.SemaphoreType.DMA] * _NSLOT
            ),
        ),
        out_shape=jax.ShapeDtypeStruct((_NROWS2, 2 * _H), jnp.float32),
    )(ids, kk, adj, s2, wfc2)


def _rate_body(feat_hbm, uidx_hbm, iidx_hbm, bfc_hbm, rating_hbm,
               uidx_v, iidx_v, uro_v, iro_v, rat_v, bfc_v, sem1, sem2):
    wid = lax.axis_index("s") * _NC + lax.axis_index("c")
    base = wid * _BW
    pltpu.sync_copy(uidx_hbm.at[pl.ds(base, _BW)], uidx_v)
    pltpu.sync_copy(iidx_hbm.at[pl.ds(base, _BW)], iidx_v)
    pltpu.sync_copy(bfc_hbm, bfc_v)
    cp1 = pltpu.async_copy(feat_hbm.at[uidx_v], uro_v, sem1)
    cp2 = pltpu.async_copy(feat_hbm.at[iidx_v], iro_v, sem2)
    cp1.wait()
    cp2.wait()
    bfc16 = bfc_v[...]

    def g_body(g, carry):
        b0 = g * 16
        bidx = b0 + lax.iota(jnp.int32, 16)
        acc = bfc16
        for f in range(_H):
            fidx = jnp.full((16,), f, jnp.int32)
            fidx2 = jnp.full((16,), _H + f, jnp.int32)
            u = plsc.load_gather(uro_v, [bidx, fidx])
            iv = plsc.load_gather(iro_v, [bidx, fidx2])
            acc = acc + u * iv
        rat_v[pl.ds(b0, 16)] = acc
        return carry

    lax.fori_loop(0, _BW // 16, g_body, 0)
    pltpu.sync_copy(rat_v, rating_hbm.at[pl.ds(base, _BW)])


_rate_call = functools.partial(
    pl.kernel,
    out_type=jax.ShapeDtypeStruct((_B,), jnp.float32),
    mesh=_sc_mesh,
    compiler_params=pltpu.CompilerParams(needs_layout_passes=False),
    scratch_types=[
        pltpu.VMEM((_BW,), jnp.int32),
        pltpu.VMEM((_BW,), jnp.int32),
        pltpu.VMEM((_BW, 2 * _H), jnp.float32),
        pltpu.VMEM((_BW, 2 * _H), jnp.float32),
        pltpu.VMEM((_BW,), jnp.float32),
        pltpu.VMEM((16,), jnp.float32),
        pltpu.SemaphoreType.DMA,
        pltpu.SemaphoreType.DMA,
    ],
)(_rate_body)


def kernel(user_indices, item_indices, adjacency_matrix, user_table,
           item_table, W1, b1, W2, b2, Wfc, bfc):
    ui = user_indices.astype(jnp.int32)
    ii = item_indices.astype(jnp.int32) + _N_USERS
    rows_all = jnp.concatenate([ui, ii])
    mask = _mask_call(rows_all)
    mask2d = mask[:_N].reshape(_N, 1)
    mi = mask[:_N].astype(jnp.int32)
    cnt = jnp.cumsum(mi)
    kk = cnt[_N - 1:]
    pos = cnt - 1
    last_id = jnp.argmax(cnt).astype(jnp.int32)
    ids = jnp.full((_NROWS2,), last_id, jnp.int32)
    ids = ids.at[jnp.where(mi == 1, pos, _NROWS2 * 4)].set(
        jnp.arange(_N, dtype=jnp.int32), mode="drop")
    s1 = _support1(user_table, item_table, W1.astype(jnp.bfloat16),
                   b1.reshape(1, _H), mask2d)
    s2 = _pass1(adjacency_matrix, s1, W2.astype(jnp.bfloat16),
                b2.reshape(1, _H))
    wfc2 = jnp.concatenate(
        [Wfc.reshape(1, _H), jnp.ones((1, _H), jnp.float32)], axis=1)
    feat = _pass2(adjacency_matrix, s2, wfc2, ids, kk)
    rating = _rate_call(feat, pos[ui], pos[ii],
                        jnp.broadcast_to(bfc, (16,)).astype(jnp.float32))
    return rating


# speedup vs baseline: 1.2650x; 1.0909x over previous
import functools

import jax
import jax.numpy as jnp
from jax import lax
from jax.experimental import pallas as pl
from jax.experimental.pallas import tpu as pltpu
from jax.experimental.pallas import tpu_sc as plsc

_N_USERS = 5000
_N_ITEMS = 5000
_N = _N_USERS + _N_ITEMS
_NPAD = 10240
_B = 4096
_F = 128
_H = 64
_NC = 2
_NS = 16
_NW = _NC * _NS
_CHUNK = _NPAD // _NW
_BW = _B // _NW

_sc_mesh = plsc.VectorSubcoreMesh(
    core_axis_name="c", subcore_axis_name="s",
    num_cores=_NC, num_subcores=_NS)


def _mask_body(rows_hbm, mask_hbm, idx_v, local_v):
    wid = lax.axis_index("s") * _NC + lax.axis_index("c")
    base = wid * _CHUNK
    zero16 = jnp.zeros((16,), jnp.float32)
    for k in range(_CHUNK // 16):
        local_v[pl.ds(k * 16, 16)] = zero16
    pltpu.sync_copy(rows_hbm, idx_v)
    ones16 = jnp.ones((16,), jnp.float32)

    def body(k, carry):
        idx = idx_v[pl.ds(k * 16, 16)]
        rel = idx - base
        inb = (rel >= 0) & (rel < _CHUNK)
        relc = jnp.clip(rel, 0, _CHUNK - 1)
        plsc.store_scatter(local_v, [relc], ones16, mask=inb)
        return carry

    lax.fori_loop(0, (2 * _B) // 16, body, 0)
    pltpu.sync_copy(local_v, mask_hbm.at[pl.ds(base, _CHUNK)])


_mask_call = functools.partial(
    pl.kernel,
    out_type=jax.ShapeDtypeStruct((_NPAD,), jnp.float32),
    mesh=_sc_mesh,
    compiler_params=pltpu.CompilerParams(needs_layout_passes=False),
    scratch_types=[
        pltpu.VMEM((2 * _B,), jnp.int32),
        pltpu.VMEM((_CHUNK,), jnp.float32),
    ],
)(_mask_body)


_RB = 400


def _pass1_body(a_ref, u_ref, i_ref, w1_ref, b1_ref, mask_ref, w2_ref,
                b2_ref, out_ref, s1_ref):
    @pl.when(pl.program_id(0) == 0)
    def _():
        dn = (((1,), (1,)), ((), ()))
        tu = lax.dot_general(u_ref[...].astype(jnp.bfloat16), w1_ref[...],
                             dn, preferred_element_type=jnp.float32)
        ti = lax.dot_general(i_ref[...].astype(jnp.bfloat16), w1_ref[...],
                             dn, preferred_element_type=jnp.float32)
        t = jnp.concatenate([tu, ti], axis=0)
        s1_ref[...] = (t * mask_ref[...] + b1_ref[...]).astype(jnp.bfloat16)

    a = a_ref[...].astype(jnp.bfloat16)
    h = lax.dot_general(a, s1_ref[...], (((1,), (0,)), ((), ())),
                        preferred_element_type=jnp.float32)
    h = jnp.maximum(h, 0.0).astype(jnp.bfloat16)
    s2 = lax.dot_general(h, w2_ref[...], (((1,), (1,)), ((), ())),
                         preferred_element_type=jnp.float32) + b2_ref[...]
    out_ref[...] = s2.astype(jnp.bfloat16)


def _pass1(adj, user_table, item_table, w1b, b1r, mask2d, w2b, b2r):
    return pl.pallas_call(
        _pass1_body,
        grid=(_N // _RB,),
        in_specs=[
            pl.BlockSpec((_RB, _N), lambda i: (i, 0)),
            pl.BlockSpec((_N_USERS, _F), lambda i: (0, 0)),
            pl.BlockSpec((_N_ITEMS, _F), lambda i: (0, 0)),
            pl.BlockSpec((_H, _F), lambda i: (0, 0)),
            pl.BlockSpec((1, _H), lambda i: (0, 0)),
            pl.BlockSpec((_N, 1), lambda i: (0, 0)),
            pl.BlockSpec((_H, _H), lambda i: (0, 0)),
            pl.BlockSpec((1, _H), lambda i: (0, 0)),
        ],
        out_specs=pl.BlockSpec((_RB, _H), lambda i: (i, 0)),
        out_shape=jax.ShapeDtypeStruct((_N, _H), jnp.bfloat16),
        scratch_shapes=[pltpu.VMEM((_N, _H), jnp.bfloat16)],
    )(adj, user_table, item_table, w1b, b1r, mask2d, w2b, b2r)


def _pass2_body(a_ref, s2_ref, wfc2_ref, out_ref):
    a = a_ref[...].astype(jnp.bfloat16)
    o = lax.dot_general(a, s2_ref[...], (((1,), (0,)), ((), ())),
                        preferred_element_type=jnp.float32)
    o = jnp.maximum(o, 0.0)
    out_ref[...] = jnp.concatenate([o, o], axis=1) * wfc2_ref[...]


def _pass2(adj, s2, wfc2):
    return pl.pallas_call(
        _pass2_body,
        grid=(_N // _RB,),
        in_specs=[
            pl.BlockSpec((_RB, _N), lambda i: (i, 0)),
            pl.BlockSpec((_N, _H), lambda i: (0, 0)),
            pl.BlockSpec((1, 2 * _H), lambda i: (0, 0)),
        ],
        out_specs=pl.BlockSpec((_RB, 2 * _H), lambda i: (i, 0)),
        out_shape=jax.ShapeDtypeStruct((_N, 2 * _H), jnp.float32),
    )(adj, s2, wfc2)


def _rate_body(feat_hbm, uidx_hbm, iidx_hbm, bfc_hbm, rating_hbm,
               uidx_v, iidx_v, uro_v, iro_v, rat_v, bfc_v, sem1, sem2):
    wid = lax.axis_index("s") * _NC + lax.axis_index("c")
    base = wid * _BW
    pltpu.sync_copy(uidx_hbm.at[pl.ds(base, _BW)], uidx_v)
    pltpu.sync_copy(iidx_hbm.at[pl.ds(base, _BW)], iidx_v)
    pltpu.sync_copy(bfc_hbm, bfc_v)
    cp1 = pltpu.async_copy(feat_hbm.at[uidx_v], uro_v, sem1)
    cp2 = pltpu.async_copy(feat_hbm.at[iidx_v], iro_v, sem2)
    cp1.wait()
    cp2.wait()
    bfc16 = bfc_v[...]

    def g_body(g, carry):
        b0 = g * 16
        bidx = b0 + lax.iota(jnp.int32, 16)
        acc = bfc16
        for f in range(_H):
            fidx = jnp.full((16,), f, jnp.int32)
            fidx2 = jnp.full((16,), _H + f, jnp.int32)
            u = plsc.load_gather(uro_v, [bidx, fidx])
            iv = plsc.load_gather(iro_v, [bidx, fidx2])
            acc = acc + u * iv
        rat_v[pl.ds(b0, 16)] = acc
        return carry

    lax.fori_loop(0, _BW // 16, g_body, 0)
    pltpu.sync_copy(rat_v, rating_hbm.at[pl.ds(base, _BW)])


_rate_call = functools.partial(
    pl.kernel,
    out_type=jax.ShapeDtypeStruct((_B,), jnp.float32),
    mesh=_sc_mesh,
    compiler_params=pltpu.CompilerParams(needs_layout_passes=False),
    scratch_types=[
        pltpu.VMEM((_BW,), jnp.int32),
        pltpu.VMEM((_BW,), jnp.int32),
        pltpu.VMEM((_BW, 2 * _H), jnp.float32),
        pltpu.VMEM((_BW, 2 * _H), jnp.float32),
        pltpu.VMEM((_BW,), jnp.float32),
        pltpu.VMEM((16,), jnp.float32),
        pltpu.SemaphoreType.DMA,
        pltpu.SemaphoreType.DMA,
    ],
)(_rate_body)


def kernel(user_indices, item_indices, adjacency_matrix, user_table,
           item_table, W1, b1, W2, b2, Wfc, bfc):
    ui = user_indices.astype(jnp.int32)
    ii = item_indices.astype(jnp.int32) + _N_USERS
    rows_all = jnp.concatenate([ui, ii])
    mask = _mask_call(rows_all)
    mask2d = mask[:_N].reshape(_N, 1)
    s2 = _pass1(adjacency_matrix, user_table, item_table,
                W1.astype(jnp.bfloat16), b1.reshape(1, _H), mask2d,
                W2.astype(jnp.bfloat16), b2.reshape(1, _H))
    wfc2 = jnp.concatenate(
        [Wfc.reshape(1, _H), jnp.ones((1, _H), jnp.float32)], axis=1)
    feat = _pass2(adjacency_matrix, s2, wfc2)
    rating = _rate_call(feat, ui, ii,
                        jnp.broadcast_to(bfc, (16,)).astype(jnp.float32))
    return rating


# speedup vs baseline: 1.2696x; 1.0036x over previous
import functools

import jax
import jax.numpy as jnp
from jax import lax
from jax.experimental import pallas as pl
from jax.experimental.pallas import tpu as pltpu
from jax.experimental.pallas import tpu_sc as plsc

_N_USERS = 5000
_N_ITEMS = 5000
_N = _N_USERS + _N_ITEMS
_NPAD = 10240
_B = 4096
_F = 128
_H = 64
_NC = 2
_NS = 16
_NW = _NC * _NS
_CHUNK = _NPAD // _NW
_BW = _B // _NW

_sc_mesh = plsc.VectorSubcoreMesh(
    core_axis_name="c", subcore_axis_name="s",
    num_cores=_NC, num_subcores=_NS)


def _mask_body(rows_hbm, mask_hbm, idx_v, local_v):
    wid = lax.axis_index("s") * _NC + lax.axis_index("c")
    base = wid * _CHUNK
    zero16 = jnp.zeros((16,), jnp.float32)
    for k in range(_CHUNK // 16):
        local_v[pl.ds(k * 16, 16)] = zero16
    pltpu.sync_copy(rows_hbm, idx_v)
    ones16 = jnp.ones((16,), jnp.float32)

    def body(k, carry):
        idx = idx_v[pl.ds(k * 16, 16)]
        rel = idx - base
        inb = (rel >= 0) & (rel < _CHUNK)
        relc = jnp.clip(rel, 0, _CHUNK - 1)
        plsc.store_scatter(local_v, [relc], ones16, mask=inb)
        return carry

    lax.fori_loop(0, (2 * _B) // 16, body, 0)
    pltpu.sync_copy(local_v, mask_hbm.at[pl.ds(base, _CHUNK)])


_mask_call = functools.partial(
    pl.kernel,
    out_type=jax.ShapeDtypeStruct((_NPAD,), jnp.float32),
    mesh=_sc_mesh,
    compiler_params=pltpu.CompilerParams(needs_layout_passes=False),
    scratch_types=[
        pltpu.VMEM((2 * _B,), jnp.int32),
        pltpu.VMEM((_CHUNK,), jnp.float32),
    ],
)(_mask_body)


_RB = 400
_NB = _N // _RB


def _gcn_body(a_ref, u_ref, i_ref, w1_ref, b1_ref, mask_ref, w2_ref,
              b2_ref, wfc2_ref, out_ref, s1_ref, s2_ref):
    i = pl.program_id(0)

    @pl.when(i == 0)
    def _():
        dn = (((1,), (1,)), ((), ()))
        tu = lax.dot_general(u_ref[...].astype(jnp.bfloat16), w1_ref[...],
                             dn, preferred_element_type=jnp.float32)
        ti = lax.dot_general(i_ref[...].astype(jnp.bfloat16), w1_ref[...],
                             dn, preferred_element_type=jnp.float32)
        t = jnp.concatenate([tu, ti], axis=0)
        s1_ref[...] = (t * mask_ref[...] + b1_ref[...]).astype(jnp.bfloat16)

    a = a_ref[...].astype(jnp.bfloat16)

    @pl.when(i < _NB)
    def _():
        h = lax.dot_general(a, s1_ref[...], (((1,), (0,)), ((), ())),
                            preferred_element_type=jnp.float32)
        h = jnp.maximum(h, 0.0).astype(jnp.bfloat16)
        s2 = lax.dot_general(h, w2_ref[...], (((1,), (1,)), ((), ())),
                             preferred_element_type=jnp.float32) + b2_ref[...]
        s2_ref[pl.ds(i * _RB, _RB), :] = s2.astype(jnp.bfloat16)

    @pl.when(i >= _NB)
    def _():
        o = lax.dot_general(a, s2_ref[...], (((1,), (0,)), ((), ())),
                            preferred_element_type=jnp.float32)
        o = jnp.maximum(o, 0.0)
        out_ref[...] = jnp.concatenate([o, o], axis=1) * wfc2_ref[...]


def _gcn(adj, user_table, item_table, w1b, b1r, mask2d, w2b, b2r, wfc2):
    def blk(i):
        return (lax.rem(i, _NB), 0)

    def full(i):
        return (0, 0)

    return pl.pallas_call(
        _gcn_body,
        grid=(2 * _NB,),
        in_specs=[
            pl.BlockSpec((_RB, _N), blk),
            pl.BlockSpec((_N_USERS, _F), full),
            pl.BlockSpec((_N_ITEMS, _F), full),
            pl.BlockSpec((_H, _F), full),
            pl.BlockSpec((1, _H), full),
            pl.BlockSpec((_N, 1), full),
            pl.BlockSpec((_H, _H), full),
            pl.BlockSpec((1, _H), full),
            pl.BlockSpec((1, 2 * _H), full),
        ],
        out_specs=pl.BlockSpec((_RB, 2 * _H), blk),
        out_shape=jax.ShapeDtypeStruct((_N, 2 * _H), jnp.float32),
        scratch_shapes=[
            pltpu.VMEM((_N, _H), jnp.bfloat16),
            pltpu.VMEM((_N, _H), jnp.bfloat16),
        ],
    )(adj, user_table, item_table, w1b, b1r, mask2d, w2b, b2r, wfc2)


def _rate_body(feat_hbm, uidx_hbm, iidx_hbm, bfc_hbm, rating_hbm,
               uidx_v, iidx_v, uro_v, iro_v, rat_v, bfc_v, sem1, sem2):
    wid = lax.axis_index("s") * _NC + lax.axis_index("c")
    base = wid * _BW
    pltpu.sync_copy(uidx_hbm.at[pl.ds(base, _BW)], uidx_v)
    pltpu.sync_copy(iidx_hbm.at[pl.ds(base, _BW)], iidx_v)
    pltpu.sync_copy(bfc_hbm, bfc_v)
    cp1 = pltpu.async_copy(feat_hbm.at[uidx_v], uro_v, sem1)
    cp2 = pltpu.async_copy(feat_hbm.at[iidx_v], iro_v, sem2)
    cp1.wait()
    cp2.wait()
    bfc16 = bfc_v[...]

    def g_body(g, carry):
        b0 = g * 16
        bidx = b0 + lax.iota(jnp.int32, 16)
        acc = bfc16
        for f in range(_H):
            fidx = jnp.full((16,), f, jnp.int32)
            fidx2 = jnp.full((16,), _H + f, jnp.int32)
            u = plsc.load_gather(uro_v, [bidx, fidx])
            iv = plsc.load_gather(iro_v, [bidx, fidx2])
            acc = acc + u * iv
        rat_v[pl.ds(b0, 16)] = acc
        return carry

    lax.fori_loop(0, _BW // 16, g_body, 0)
    pltpu.sync_copy(rat_v, rating_hbm.at[pl.ds(base, _BW)])


_rate_call = functools.partial(
    pl.kernel,
    out_type=jax.ShapeDtypeStruct((_B,), jnp.float32),
    mesh=_sc_mesh,
    compiler_params=pltpu.CompilerParams(needs_layout_passes=False),
    scratch_types=[
        pltpu.VMEM((_BW,), jnp.int32),
        pltpu.VMEM((_BW,), jnp.int32),
        pltpu.VMEM((_BW, 2 * _H), jnp.float32),
        pltpu.VMEM((_BW, 2 * _H), jnp.float32),
        pltpu.VMEM((_BW,), jnp.float32),
        pltpu.VMEM((16,), jnp.float32),
        pltpu.SemaphoreType.DMA,
        pltpu.SemaphoreType.DMA,
    ],
)(_rate_body)


def kernel(user_indices, item_indices, adjacency_matrix, user_table,
           item_table, W1, b1, W2, b2, Wfc, bfc):
    ui = user_indices.astype(jnp.int32)
    ii = item_indices.astype(jnp.int32) + _N_USERS
    rows_all = jnp.concatenate([ui, ii])
    mask = _mask_call(rows_all)
    mask2d = mask[:_N].reshape(_N, 1)
    wfc2 = jnp.concatenate(
        [Wfc.reshape(1, _H), jnp.ones((1, _H), jnp.float32)], axis=1)
    feat = _gcn(adjacency_matrix, user_table, item_table,
                W1.astype(jnp.bfloat16), b1.reshape(1, _H), mask2d,
                W2.astype(jnp.bfloat16), b2.reshape(1, _H), wfc2)
    rating = _rate_call(feat, ui, ii,
                        jnp.broadcast_to(bfc, (16,)).astype(jnp.float32))
    return rating
